# Initial kernel scaffold; baseline (speedup 1.0000x reference)
import jax
import jax.numpy as jnp
from jax.experimental import pallas as pl


def kernel(node_feats, edge_index, edge_types, rel_w1, lin_w1, lin_b1, rel_w2, lin_w2, lin_b2):
    raise NotImplementedError("write your pallas kernel here")


# speedup vs baseline: 1.9818x; 1.9818x over previous
import functools

import jax
import jax.numpy as jnp
from jax import lax
from jax.experimental import pallas as pl
from jax.experimental.pallas import tpu as pltpu
from jax.experimental.pallas import tpu_sc as plsc

N = 10000
E = 320000
D = 128
DE = 144
R = 16

_INFO = plsc.get_sparse_core_info()
NC = _INFO.num_cores
NS = _INFO.num_subcores
NW = NC * NS
EW = E // NW
B = 80
NBLK = EW // B
NP = 10240
RPT = NP // NS


def _rel_matmul(x, w, wide):
    nb = 5
    bn = N // nb
    wout = DE if wide else D

    def body(x_ref, w_ref, o_ref):
        h = jnp.dot(x_ref[...], w_ref[0], preferred_element_type=jnp.float32)
        if wide:
            h = jnp.concatenate(
                [h, jnp.ones((bn, DE - D), jnp.float32)], axis=1)
        o_ref[0] = h

    return pl.pallas_call(
        body,
        grid=(nb, w.shape[0]),
        in_specs=[
            pl.BlockSpec((bn, D), lambda i, r: (i, 0)),
            pl.BlockSpec((1, D, D), lambda i, r: (r, 0, 0)),
        ],
        out_specs=pl.BlockSpec((1, bn, wout), lambda i, r: (r, i, 0)),
        out_shape=jax.ShapeDtypeStruct((w.shape[0], N, wout), jnp.float32),
    )(x, w)


def _finish1(p, lin, b):
    nb = 5
    bn = N // nb

    def body(p_ref, lin_ref, b_ref, o_ref, inv_ref):
        s = p_ref[0] + p_ref[1]
        inv = 1.0 / jnp.maximum(s[:, D:D + 1], 1.0)
        o_ref[...] = jnp.tanh(s[:, :D] * inv + lin_ref[:, :D] + b_ref[...])
        inv_ref[...] = inv

    return pl.pallas_call(
        body,
        grid=(nb,),
        in_specs=[
            pl.BlockSpec((2, bn, DE), lambda i: (0, i, 0)),
            pl.BlockSpec((bn, DE), lambda i: (i, 0)),
            pl.BlockSpec((1, D), lambda i: (0, 0)),
        ],
        out_specs=[
            pl.BlockSpec((bn, D), lambda i: (i, 0)),
            pl.BlockSpec((bn, 1), lambda i: (i, 0)),
        ],
        out_shape=[
            jax.ShapeDtypeStruct((N, D), jnp.float32),
            jax.ShapeDtypeStruct((N, 1), jnp.float32),
        ],
    )(p, lin, b.reshape(1, D))


def _finish2(p, inv, lin, b):
    nb = 5
    bn = N // nb

    def body(p_ref, inv_ref, lin_ref, b_ref, o_ref):
        s = p_ref[0] + p_ref[1]
        o_ref[...] = jnp.tanh(s * inv_ref[...] + lin_ref[...] + b_ref[...])

    return pl.pallas_call(
        body,
        grid=(nb,),
        in_specs=[
            pl.BlockSpec((2, bn, D), lambda i: (0, i, 0)),
            pl.BlockSpec((bn, 1), lambda i: (i, 0)),
            pl.BlockSpec((bn, D), lambda i: (i, 0)),
            pl.BlockSpec((1, D), lambda i: (0, 0)),
        ],
        out_specs=pl.BlockSpec((bn, D), lambda i: (i, 0)),
        out_shape=jax.ShapeDtypeStruct((N, D), jnp.float32),
    )(p, inv, lin, b.reshape(1, D))


def _make_sc_agg(w):
    mesh = plsc.VectorSubcoreMesh(core_axis_name="c", subcore_axis_name="s")
    out_type = jax.ShapeDtypeStruct((NC, NP, w), jnp.float32)
    scratch = [
        pltpu.VMEM((B,), jnp.int32),
        pltpu.VMEM((B,), jnp.int32),
        pltpu.VMEM((B, w), jnp.float32),
        pltpu.VMEM_SHARED((NP, w), jnp.float32),
        pltpu.SemaphoreType.DMA,
    ]

    def body(table, cidx, dst, zrow, out, idx_v, dst_v, rows_v, acc_sh, sem):
        c = lax.axis_index("c")
        s = lax.axis_index("s")
        wid = s * NC + c
        r0 = s * RPT
        nchunk = RPT // B

        pltpu.sync_copy(zrow, rows_v)
        for k in range(nchunk):
            pltpu.sync_copy(rows_v, acc_sh.at[pl.ds(r0 + k * B, B)])
        plsc.subcore_barrier()

        e0 = wid * EW

        def blk(bi, carry):
            off = e0 + bi * B
            pltpu.sync_copy(cidx.at[pl.ds(off, B)], idx_v)
            pltpu.sync_copy(dst.at[pl.ds(off, B)], dst_v)
            pltpu.async_copy(table.at[idx_v], rows_v, sem).wait()
            pltpu.sync_copy(rows_v, acc_sh.at[dst_v], add=True)
            return carry

        lax.fori_loop(0, NBLK, blk, 0)
        plsc.subcore_barrier()

        for k in range(nchunk):
            pltpu.sync_copy(acc_sh.at[pl.ds(r0 + k * B, B)], rows_v)
            pltpu.sync_copy(rows_v, out.at[c, pl.ds(r0 + k * B, B)])

    return pl.kernel(
        body, out_type=out_type, mesh=mesh, scratch_types=scratch,
        compiler_params=pltpu.CompilerParams(use_tc_tiling_on_sc=False))


_SC_AGG_WIDE = _make_sc_agg(DE)
_SC_AGG = _make_sc_agg(D)


def kernel(node_feats, edge_index, edge_types, rel_w1, lin_w1, lin_b1,
           rel_w2, lin_w2, lin_b2):
    src = edge_index[0].astype(jnp.int32)
    dst = edge_index[1].astype(jnp.int32)
    et = edge_types.astype(jnp.int32)
    cidx = et * N + src

    zDE = jnp.zeros((B, DE), jnp.float32)
    zD = jnp.zeros((B, D), jnp.float32)

    w1e = jnp.concatenate([rel_w1, lin_w1[None]], axis=0)
    w2e = jnp.concatenate([rel_w2, lin_w2[None]], axis=0)

    hr1 = _rel_matmul(node_feats, w1e, wide=True)
    p1 = _SC_AGG_WIDE(hr1[:R].reshape(R * N, DE), cidx, dst, zDE)
    h1, inv = _finish1(p1, hr1[R], lin_b1)

    hr2 = _rel_matmul(h1, w2e, wide=False)
    p2 = _SC_AGG(hr2[:R].reshape(R * N, D), cidx, dst, zD)
    h2 = _finish2(p2, inv, hr2[R], lin_b2)
    return h2
